# baseline (speedup 1.0000x reference)
import functools

import jax
import jax.numpy as jnp
from jax import lax
from jax.experimental import pallas as pl
from jax.experimental.pallas import tpu as pltpu
from jax.experimental.pallas import tpu_sc as plsc

N = 10000
E = 320000
D = 128

NC = 2
NS = 16
NW = NC * NS
EW = E // NW
CH = 80
NCH = EW // CH
NPAD = 10240
RPT = NPAD // NS
EWP = 10112
HB = 48
NH = 210
EWA = NH * HB

_mesh = plsc.VectorSubcoreMesh(core_axis_name="c", subcore_axis_name="s")


@functools.partial(
    pl.kernel,
    out_type=jax.ShapeDtypeStruct((NC, NPAD), jnp.float32),
    mesh=_mesh,
    scratch_types=[
        pltpu.VMEM((NCH, CH), jnp.int32),
        pltpu.VMEM((NCH, CH), jnp.float32),
        pltpu.VMEM_SHARED((NPAD,), jnp.float32),
    ],
)
def _deg_kernel(dst_hbm, ew_hbm, zn_hbm, out_hbm, dst_v, ew_v, deg_sh):
    c = lax.axis_index("c")
    s = lax.axis_index("s")
    w = c * NS + s
    pltpu.sync_copy(zn_hbm, deg_sh.at[pl.ds(s * RPT, RPT)])
    pltpu.sync_copy(dst_hbm.at[w], dst_v)
    pltpu.sync_copy(ew_hbm.at[w], ew_v)
    plsc.subcore_barrier()

    def body(j, carry):
        pltpu.sync_copy(ew_v.at[j], deg_sh.at[dst_v.at[j]], add=True)
        return carry

    lax.fori_loop(0, NCH, body, 0)
    plsc.subcore_barrier()
    pltpu.sync_copy(deg_sh.at[pl.ds(s * RPT, RPT)],
                    out_hbm.at[c, pl.ds(s * RPT, RPT)])


@functools.partial(
    pl.kernel,
    out_type=jax.ShapeDtypeStruct((NC, NPAD, D), jnp.float32),
    mesh=_mesh,
    scratch_types=[
        pltpu.VMEM((EWP,), jnp.int32),
        pltpu.VMEM((EWP,), jnp.float32),
        pltpu.VMEM((HB,), jnp.int32),
        pltpu.VMEM((HB,), jnp.int32),
        pltpu.VMEM((HB,), jnp.int32),
        pltpu.VMEM((HB,), jnp.int32),
        pltpu.VMEM((HB, D), jnp.float32),
        pltpu.VMEM((HB, D), jnp.float32),
        pltpu.VMEM_SHARED((NPAD, D), jnp.float32),
        pltpu.SemaphoreType.DMA,
        pltpu.SemaphoreType.DMA,
    ],
)
def _agg_kernel(y_hbm, se_hbm, ew_hbm, z2_hbm, out_hbm,
                se_v, ew_v, sa, da, sb, db, gba, gbb, acc_sh, sema, semb):
    c = lax.axis_index("c")
    s = lax.axis_index("s")
    w = c * NS + s
    pltpu.sync_copy(z2_hbm, acc_sh.at[pl.ds(s * RPT, RPT)])
    pltpu.sync_copy(se_hbm.at[w], se_v)
    pltpu.sync_copy(ew_hbm.at[w], ew_v)
    plsc.subcore_barrier()

    lane0 = jnp.zeros((16,), jnp.int32)
    gdn = lax.GatherDimensionNumbers(
        offset_dims=(), collapsed_slice_dims=(0,), start_index_map=(0,))

    def unpack(h, sc, dc):
        @plsc.parallel_loop(0, HB // 16)
        def _(g):
            pv = se_v[pl.ds(h * HB + g * 16, 16)]
            sl = pl.ds(g * 16, 16)
            sc[sl] = lax.bitwise_and(pv, 0xFFFF)
            dc[sl] = lax.shift_right_logical(pv, 16)

    def gather_start(sc, gb, sem):
        pltpu.async_copy(y_hbm.at[sc], gb, sem)

    def gather_wait(sc, gb, sem):
        pltpu.make_async_copy(y_hbm.at[sc], gb, sem).wait()

    def scale(h, gb):
        @plsc.parallel_loop(0, HB, unroll=2)
        def _(e):
            wv16 = ew_v[pl.ds(h * HB + e, 16)]
            wv = lax.gather(wv16, lane0[:, None], dimension_numbers=gdn,
                            slice_sizes=(1,),
                            mode=lax.GatherScatterMode.PROMISE_IN_BOUNDS)
            for v in range(D // 16):
                sl = pl.ds(v * 16, 16)
                gb[e, sl] = gb[e, sl] * wv

    unpack(0, sa, da)
    gather_start(sa, gba, sema)

    def pair(t, carry):
        h = 2 * t
        unpack(h + 1, sb, db)
        gather_start(sb, gbb, semb)
        gather_wait(sa, gba, sema)
        scale(h, gba)
        pltpu.sync_copy(gba, acc_sh.at[da], add=True)

        @pl.when(t < NH // 2 - 1)
        def _():
            unpack(h + 2, sa, da)
            gather_start(sa, gba, sema)

        gather_wait(sb, gbb, semb)
        scale(h + 1, gbb)
        pltpu.sync_copy(gbb, acc_sh.at[db], add=True)
        return carry

    lax.fori_loop(0, NH // 2, pair, 0)
    plsc.subcore_barrier()
    pltpu.sync_copy(acc_sh.at[pl.ds(s * RPT, RPT)],
                    out_hbm.at[c, pl.ds(s * RPT, RPT)])


_BN = 2000


def _prep_body(dp_ref, x_ref, w_ref, y_ref, dinv_ref):
    deg = 1.0 + dp_ref[:, 0:1] + dp_ref[:, 1:2]
    dinv = lax.rsqrt(deg)
    xw = jnp.dot(x_ref[:, :], w_ref[:, :], preferred_element_type=jnp.float32)
    y_ref[:, :] = dinv * xw
    dinv_ref[:, :] = dinv


def _mid_body(p0_ref, p1_ref, y0_ref, dinv_ref, b_ref, w_ref, y1_ref):
    dinv = dinv_ref[:, :]
    h = dinv * (p0_ref[:, :] + p1_ref[:, :] + y0_ref[:, :]) + b_ref[:, :]
    h = jnp.maximum(h, 0.0)
    y1_ref[:, :] = dinv * jnp.dot(h, w_ref[:, :],
                                  preferred_element_type=jnp.float32)


def _fin_body(q0_ref, q1_ref, y1_ref, dinv_ref, b_ref, out_ref):
    out_ref[:, :] = (dinv_ref[:, :]
                     * (q0_ref[:, :] + q1_ref[:, :] + y1_ref[:, :])
                     + b_ref[:, :])


def _row_spec(width):
    return pl.BlockSpec((_BN, width), lambda i: (i, 0))


def _full_spec(shape):
    return pl.BlockSpec(shape, lambda i: (0, 0))


def _prep(dp_t, x, W0):
    return pl.pallas_call(
        _prep_body,
        grid=(N // _BN,),
        in_specs=[_row_spec(2), _row_spec(D), _full_spec((D, D))],
        out_specs=[_row_spec(D), _row_spec(1)],
        out_shape=[jax.ShapeDtypeStruct((N, D), jnp.float32),
                   jax.ShapeDtypeStruct((N, 1), jnp.float32)],
    )(dp_t, x, W0)


def _mid(p0, p1, y0, dinv, b0, W1):
    return pl.pallas_call(
        _mid_body,
        grid=(N // _BN,),
        in_specs=[_row_spec(D), _row_spec(D), _row_spec(D), _row_spec(1),
                  _full_spec((1, D)), _full_spec((D, D))],
        out_specs=_row_spec(D),
        out_shape=jax.ShapeDtypeStruct((N, D), jnp.float32),
    )(p0, p1, y0, dinv, b0, W1)


def _fin(q0, q1, y1, dinv, b1):
    return pl.pallas_call(
        _fin_body,
        grid=(N // _BN,),
        in_specs=[_row_spec(D), _row_spec(D), _row_spec(D), _row_spec(1),
                  _full_spec((1, D))],
        out_specs=_row_spec(D),
        out_shape=jax.ShapeDtypeStruct((N, D), jnp.float32),
    )(q0, q1, y1, dinv, b1)


@jax.jit
def kernel(x, edge_index_0, edge_weight_0, W0, b0, W1, b1):
    src = edge_index_0[0]
    dst = edge_index_0[1]
    ew = edge_weight_0.reshape(NW, NCH, CH)
    wsub = (jnp.arange(NW, dtype=jnp.int32) % NS)[:, None]
    pad_dst = N + wsub * 15 + (jnp.arange(EWA - EW, dtype=jnp.int32) % 15)
    se = jnp.concatenate(
        [(lax.shift_left(dst, 16) | src).reshape(NW, EW),
         lax.shift_left(pad_dst, 16),
         jnp.zeros((NW, EWP - EWA), jnp.int32)], axis=1)
    ew_flat = jnp.concatenate(
        [edge_weight_0.reshape(NW, EW),
         jnp.zeros((NW, EWP - EW), jnp.float32)], axis=1)
    zn = jnp.zeros((RPT,), jnp.float32)
    z2 = jnp.zeros((RPT, D), jnp.float32)

    degp = _deg_kernel(dst.reshape(NW, NCH, CH), ew, zn)
    dp_t = degp[:, :N].T
    y0, dinv = _prep(dp_t, x, W0)

    p = _agg_kernel(y0, se, ew_flat, z2)
    y1 = _mid(p[0, :N], p[1, :N], y0, dinv, b0.reshape(1, D), W1)

    q = _agg_kernel(y1, se, ew_flat, z2)
    return _fin(q[0, :N], q[1, :N], y1, dinv, b1.reshape(1, D))


# speedup vs baseline: 1.0032x; 1.0032x over previous
import functools

import jax
import jax.numpy as jnp
from jax import lax
from jax.experimental import pallas as pl
from jax.experimental.pallas import tpu as pltpu
from jax.experimental.pallas import tpu_sc as plsc

N = 10000
E = 320000
D = 128

NC = 2
NS = 16
NW = NC * NS
EW = E // NW
CH = 80
NCH = EW // CH
NPAD = 10240
RPT = NPAD // NS
EWP = 10112
NR = 2
EPR = EW // NR
EPRP = 5120
RCH = EPRP // CH
EWB = 5248

_mesh = plsc.VectorSubcoreMesh(core_axis_name="c", subcore_axis_name="s")


@functools.partial(
    pl.kernel,
    out_type=jax.ShapeDtypeStruct((NC, NPAD), jnp.float32),
    mesh=_mesh,
    scratch_types=[
        pltpu.VMEM((NCH, CH), jnp.int32),
        pltpu.VMEM((NCH, CH), jnp.float32),
        pltpu.VMEM_SHARED((NPAD,), jnp.float32),
    ],
)
def _deg_kernel(dst_hbm, ew_hbm, zn_hbm, out_hbm, dst_v, ew_v, deg_sh):
    c = lax.axis_index("c")
    s = lax.axis_index("s")
    w = c * NS + s
    pltpu.sync_copy(zn_hbm, deg_sh.at[pl.ds(s * RPT, RPT)])
    pltpu.sync_copy(dst_hbm.at[w], dst_v)
    pltpu.sync_copy(ew_hbm.at[w], ew_v)
    plsc.subcore_barrier()

    def body(j, carry):
        pltpu.sync_copy(ew_v.at[j], deg_sh.at[dst_v.at[j]], add=True)
        return carry

    lax.fori_loop(0, NCH, body, 0)
    plsc.subcore_barrier()
    pltpu.sync_copy(deg_sh.at[pl.ds(s * RPT, RPT)],
                    out_hbm.at[c, pl.ds(s * RPT, RPT)])


@functools.partial(
    pl.kernel,
    out_type=jax.ShapeDtypeStruct((NC, NPAD, D), jnp.float32),
    mesh=_mesh,
    scratch_types=[
        pltpu.VMEM((EWP,), jnp.int32),
        pltpu.VMEM((EWP,), jnp.float32),
        pltpu.VMEM((CH,), jnp.int32),
        pltpu.VMEM((CH,), jnp.int32),
        pltpu.VMEM((CH, D), jnp.float32),
        pltpu.VMEM_SHARED((NPAD, D), jnp.float32),
    ],
)
def _agg_kernel(y_hbm, se_hbm, ew_hbm, z2_hbm, out_hbm,
                se_v, ew_v, src_c, dst_c, gbuf, acc_sh):
    c = lax.axis_index("c")
    s = lax.axis_index("s")
    w = c * NS + s
    pltpu.sync_copy(z2_hbm, acc_sh.at[pl.ds(s * RPT, RPT)])
    pltpu.sync_copy(se_hbm.at[w], se_v)
    pltpu.sync_copy(ew_hbm.at[w], ew_v)
    plsc.subcore_barrier()

    lane0 = jnp.zeros((16,), jnp.int32)
    gdn = lax.GatherDimensionNumbers(
        offset_dims=(), collapsed_slice_dims=(0,), start_index_map=(0,))

    def chunk(j, carry):
        @plsc.parallel_loop(0, CH // 16)
        def unpack(g):
            pv = se_v[pl.ds(j * CH + g * 16, 16)]
            sl = pl.ds(g * 16, 16)
            src_c[sl] = lax.bitwise_and(pv, 0xFFFF)
            dst_c[sl] = lax.shift_right_logical(pv, 16)
        pltpu.sync_copy(y_hbm.at[src_c], gbuf)

        @plsc.parallel_loop(0, CH, unroll=2)
        def scale(e):
            wv16 = ew_v[pl.ds(j * CH + e, 16)]
            wv = lax.gather(wv16, lane0[:, None], dimension_numbers=gdn,
                            slice_sizes=(1,),
                            mode=lax.GatherScatterMode.PROMISE_IN_BOUNDS)
            for v in range(D // 16):
                sl = pl.ds(v * 16, 16)
                gbuf[e, sl] = gbuf[e, sl] * wv
        pltpu.sync_copy(gbuf, acc_sh.at[dst_c], add=True)
        return carry

    lax.fori_loop(0, NCH, chunk, 0)
    plsc.subcore_barrier()
    pltpu.sync_copy(acc_sh.at[pl.ds(s * RPT, RPT)],
                    out_hbm.at[c, pl.ds(s * RPT, RPT)])


_BN = 2000


def _prep_body(dp_ref, x_ref, w_ref, y_ref, dinv_ref):
    deg = 1.0 + dp_ref[:, 0:1] + dp_ref[:, 1:2]
    dinv = lax.rsqrt(deg)
    xw = jnp.dot(x_ref[:, :], w_ref[:, :], preferred_element_type=jnp.float32)
    y_ref[:, :] = dinv * xw
    dinv_ref[:, :] = dinv


def _mid_body(p0_ref, p1_ref, y0_ref, dinv_ref, b_ref, w_ref, y1_ref):
    dinv = dinv_ref[:, :]
    h = dinv * (p0_ref[:, :] + p1_ref[:, :] + y0_ref[:, :]) + b_ref[:, :]
    h = jnp.maximum(h, 0.0)
    y1_ref[:, :] = dinv * jnp.dot(h, w_ref[:, :],
                                  preferred_element_type=jnp.float32)


def _fin_body(q0_ref, q1_ref, y1_ref, dinv_ref, b_ref, out_ref):
    out_ref[:, :] = (dinv_ref[:, :]
                     * (q0_ref[:, :] + q1_ref[:, :] + y1_ref[:, :])
                     + b_ref[:, :])


def _row_spec(width):
    return pl.BlockSpec((_BN, width), lambda i: (i, 0))


def _full_spec(shape):
    return pl.BlockSpec(shape, lambda i: (0, 0))


def _prep(dp_t, x, W0):
    return pl.pallas_call(
        _prep_body,
        grid=(N // _BN,),
        in_specs=[_row_spec(2), _row_spec(D), _full_spec((D, D))],
        out_specs=[_row_spec(D), _row_spec(1)],
        out_shape=[jax.ShapeDtypeStruct((N, D), jnp.float32),
                   jax.ShapeDtypeStruct((N, 1), jnp.float32)],
    )(dp_t, x, W0)


def _mid(p0, p1, y0, dinv, b0, W1):
    return pl.pallas_call(
        _mid_body,
        grid=(N // _BN,),
        in_specs=[_row_spec(D), _row_spec(D), _row_spec(D), _row_spec(1),
                  _full_spec((1, D)), _full_spec((D, D))],
        out_specs=_row_spec(D),
        out_shape=jax.ShapeDtypeStruct((N, D), jnp.float32),
    )(p0, p1, y0, dinv, b0, W1)


def _fin(q0, q1, y1, dinv, b1):
    return pl.pallas_call(
        _fin_body,
        grid=(N // _BN,),
        in_specs=[_row_spec(D), _row_spec(D), _row_spec(D), _row_spec(1),
                  _full_spec((1, D))],
        out_specs=_row_spec(D),
        out_shape=jax.ShapeDtypeStruct((N, D), jnp.float32),
    )(q0, q1, y1, dinv, b1)


@jax.jit
def kernel(x, edge_index_0, edge_weight_0, W0, b0, W1, b1):
    src = edge_index_0[0]
    dst = edge_index_0[1]
    ew = edge_weight_0.reshape(NW, NCH, CH)
    se = jnp.concatenate(
        [(lax.shift_left(dst, 16) | src).reshape(NW, EW),
         jnp.zeros((NW, EWP - EW), jnp.int32)], axis=1)
    ew_flat = jnp.concatenate(
        [edge_weight_0.reshape(NW, EW),
         jnp.zeros((NW, EWP - EW), jnp.float32)], axis=1)
    zn = jnp.zeros((RPT,), jnp.float32)
    z2 = jnp.zeros((RPT, D), jnp.float32)

    degp = _deg_kernel(dst.reshape(NW, NCH, CH), ew, zn)
    dp_t = degp[:, :N].T
    y0, dinv = _prep(dp_t, x, W0)

    p = _agg_kernel(y0, se, ew_flat, z2)
    y1 = _mid(p[0, :N], p[1, :N], y0, dinv, b0.reshape(1, D), W1)

    q = _agg_kernel(y1, se, ew_flat, z2)
    return _fin(q[0, :N], q[1, :N], y1, dinv, b1.reshape(1, D))


# speedup vs baseline: 1.0047x; 1.0015x over previous
import functools

import jax
import jax.numpy as jnp
from jax import lax
from jax.experimental import pallas as pl
from jax.experimental.pallas import tpu as pltpu
from jax.experimental.pallas import tpu_sc as plsc

N = 10000
E = 320000
D = 128

NC = 2
NS = 16
NW = NC * NS
EW = E // NW
CH = 80
NCH = EW // CH
NPAD = 10240
RPT = NPAD // NS
EWP = 10112

_mesh = plsc.VectorSubcoreMesh(core_axis_name="c", subcore_axis_name="s")


@functools.partial(
    pl.kernel,
    out_type=jax.ShapeDtypeStruct((NC, NPAD), jnp.float32),
    mesh=_mesh,
    scratch_types=[
        pltpu.VMEM((NCH, CH), jnp.int32),
        pltpu.VMEM((NCH, CH), jnp.float32),
        pltpu.VMEM_SHARED((NPAD,), jnp.float32),
    ],
)
def _deg_kernel(dst_hbm, ew_hbm, zn_hbm, out_hbm, dst_v, ew_v, deg_sh):
    c = lax.axis_index("c")
    s = lax.axis_index("s")
    w = c * NS + s
    pltpu.sync_copy(zn_hbm, deg_sh.at[pl.ds(s * RPT, RPT)])
    pltpu.sync_copy(dst_hbm.at[w], dst_v)
    pltpu.sync_copy(ew_hbm.at[w], ew_v)
    plsc.subcore_barrier()

    def body(j, carry):
        pltpu.sync_copy(ew_v.at[j], deg_sh.at[dst_v.at[j]], add=True)
        return carry

    lax.fori_loop(0, NCH, body, 0)
    plsc.subcore_barrier()
    pltpu.sync_copy(deg_sh.at[pl.ds(s * RPT, RPT)],
                    out_hbm.at[c, pl.ds(s * RPT, RPT)])


@functools.partial(
    pl.kernel,
    out_type=jax.ShapeDtypeStruct((NC, NPAD, D), jnp.float32),
    mesh=_mesh,
    scratch_types=[
        pltpu.VMEM((EWP,), jnp.int32),
        pltpu.VMEM((EWP,), jnp.float32),
        pltpu.VMEM((CH,), jnp.int32),
        pltpu.VMEM((CH,), jnp.int32),
        pltpu.VMEM((CH, D), jnp.float32),
        pltpu.VMEM_SHARED((NPAD, D), jnp.float32),
    ],
)
def _agg_kernel(y_hbm, se_hbm, ew_hbm, z2_hbm, out_hbm,
                se_v, ew_v, src_c, dst_c, gbuf, acc_sh):
    c = lax.axis_index("c")
    s = lax.axis_index("s")
    w = c * NS + s
    pltpu.sync_copy(z2_hbm, acc_sh.at[pl.ds(s * RPT, RPT)])
    pltpu.sync_copy(se_hbm.at[w], se_v)
    pltpu.sync_copy(ew_hbm.at[w], ew_v)
    plsc.subcore_barrier()

    lane0 = jnp.zeros((16,), jnp.int32)
    gdn = lax.GatherDimensionNumbers(
        offset_dims=(), collapsed_slice_dims=(0,), start_index_map=(0,))

    def chunk(j, carry):
        @plsc.parallel_loop(0, CH // 16)
        def unpack(g):
            pv = se_v[pl.ds(j * CH + g * 16, 16)]
            sl = pl.ds(g * 16, 16)
            src_c[sl] = lax.bitwise_and(pv, 0xFFFF)
            dst_c[sl] = lax.shift_right_logical(pv, 16)
        pltpu.sync_copy(y_hbm.at[src_c], gbuf)

        @plsc.parallel_loop(0, CH, unroll=2)
        def scale(e):
            wv16 = ew_v[pl.ds(j * CH + e, 16)]
            wv = lax.gather(wv16, lane0[:, None], dimension_numbers=gdn,
                            slice_sizes=(1,),
                            mode=lax.GatherScatterMode.PROMISE_IN_BOUNDS)
            for v in range(D // 16):
                sl = pl.ds(v * 16, 16)
                gbuf[e, sl] = gbuf[e, sl] * wv
        pltpu.sync_copy(gbuf, acc_sh.at[dst_c], add=True)
        return carry

    lax.fori_loop(0, NCH, chunk, 0)
    plsc.subcore_barrier()
    pltpu.sync_copy(acc_sh.at[pl.ds(s * RPT, RPT)],
                    out_hbm.at[c, pl.ds(s * RPT, RPT)])


_BN = 2000


def _prep_body(dp_ref, x_ref, w_ref, y_ref, dinv_ref):
    deg = 1.0 + dp_ref[:, 0:1] + dp_ref[:, 1:2]
    dinv = lax.rsqrt(deg)
    xw = jnp.dot(x_ref[:, :], w_ref[:, :], preferred_element_type=jnp.float32)
    y_ref[:, :] = dinv * xw
    dinv_ref[:, :] = dinv


def _mid_body(p0_ref, p1_ref, y0_ref, dinv_ref, b_ref, w_ref, y1_ref):
    dinv = dinv_ref[:, :]
    h = dinv * (p0_ref[:, :] + p1_ref[:, :] + y0_ref[:, :]) + b_ref[:, :]
    h = jnp.maximum(h, 0.0)
    y1_ref[:, :] = dinv * jnp.dot(h, w_ref[:, :],
                                  preferred_element_type=jnp.float32)


def _fin_body(q0_ref, q1_ref, y1_ref, dinv_ref, b_ref, out_ref):
    out_ref[:, :] = (dinv_ref[:, :]
                     * (q0_ref[:, :] + q1_ref[:, :] + y1_ref[:, :])
                     + b_ref[:, :])


def _row_spec(width):
    return pl.BlockSpec((_BN, width), lambda i: (i, 0))


def _full_spec(shape):
    return pl.BlockSpec(shape, lambda i: (0, 0))


def _prep(dp_t, x, W0):
    return pl.pallas_call(
        _prep_body,
        grid=(N // _BN,),
        in_specs=[_row_spec(2), _row_spec(D), _full_spec((D, D))],
        out_specs=[_row_spec(D), _row_spec(1)],
        out_shape=[jax.ShapeDtypeStruct((N, D), jnp.float32),
                   jax.ShapeDtypeStruct((N, 1), jnp.float32)],
    )(dp_t, x, W0)


def _mid(p0, p1, y0, dinv, b0, W1):
    return pl.pallas_call(
        _mid_body,
        grid=(N // _BN,),
        in_specs=[_row_spec(D), _row_spec(D), _row_spec(D), _row_spec(1),
                  _full_spec((1, D)), _full_spec((D, D))],
        out_specs=_row_spec(D),
        out_shape=jax.ShapeDtypeStruct((N, D), jnp.float32),
    )(p0, p1, y0, dinv, b0, W1)


def _fin(q0, q1, y1, dinv, b1):
    return pl.pallas_call(
        _fin_body,
        grid=(N // _BN,),
        in_specs=[_row_spec(D), _row_spec(D), _row_spec(D), _row_spec(1),
                  _full_spec((1, D))],
        out_specs=_row_spec(D),
        out_shape=jax.ShapeDtypeStruct((N, D), jnp.float32),
    )(q0, q1, y1, dinv, b1)


@jax.jit
def kernel(x, edge_index_0, edge_weight_0, W0, b0, W1, b1):
    src = edge_index_0[0]
    dst = edge_index_0[1]
    ew = edge_weight_0.reshape(NW, NCH, CH)
    se = jnp.concatenate(
        [(lax.shift_left(dst, 16) | src).reshape(NW, EW),
         jnp.zeros((NW, EWP - EW), jnp.int32)], axis=1)
    ew_flat = jnp.concatenate(
        [edge_weight_0.reshape(NW, EW),
         jnp.zeros((NW, EWP - EW), jnp.float32)], axis=1)
    zn = jnp.zeros((RPT,), jnp.float32)
    z2 = jnp.zeros((RPT, D), jnp.float32)

    degp = _deg_kernel(dst.reshape(NW, NCH, CH), ew, zn)
    dp_t = degp[:, :N].T
    y0, dinv = _prep(dp_t, x, W0)

    p = _agg_kernel(y0, se, ew_flat, z2)
    y1 = _mid(p[0, :N], p[1, :N], y0, dinv, b0.reshape(1, D), W1)

    q = _agg_kernel(y1, se, ew_flat, z2)
    return _fin(q[0, :N], q[1, :N], y1, dinv, b1.reshape(1, D))
